# baseline (speedup 1.0000x reference)
import functools

import jax
import jax.numpy as jnp
from jax import lax
from jax.experimental import pallas as pl
from jax.experimental.pallas import tpu as pltpu
from jax.experimental.pallas import tpu_sc as plsc

N_NODES = 10000
N_EDGES = 320000
D = 128

NC = 2
NS = 16
NW = NC * NS

N_PAD = 10112
ROWS_PER_TILE = N_PAD // NS
LANES = 128
CHUNK = 64
NB = 4
S0 = 64
S1 = 256
E_PAD = (S0 + S1) * CHUNK * NS
IDX_MAX = S1 * CHUNK // LANES
_WB = [CHUNK] * 9 + [ROWS_PER_TILE - 9 * CHUNK]

_DST_SHIFT = 14


def _sc_body(x_hbm, packed_hbm, out_hbm, idx_p, srcb, dstb, rows, agg, *gsems):
    c = lax.axis_index("c")
    s = lax.axis_index("s")

    steps = jnp.where(c == 0, S0, S1)
    idx_rows = lax.shift_right_logical(steps, 1)
    base = jnp.where(c == 0, s * (S0 // 2), NS * (S0 // 2) + s * (S1 // 2))

    for q in range(IDX_MAX // 32):

        @pl.when(q * 32 < idx_rows)
        def _():
            pltpu.sync_copy(
                packed_hbm.at[pl.ds(base + q * 32, 32)],
                idx_p.at[pl.ds(q * 32, 32)],
            )

    z = jnp.zeros((16,), jnp.float32)

    def _zero_row(i, _):
        for k in range(8):
            rows[0, i, pl.ds(k * 16, 16)] = z
        return 0

    lax.fori_loop(0, CHUNK, _zero_row, 0)
    rbase = s * ROWS_PER_TILE
    off = 0
    for wb in _WB:
        pltpu.sync_copy(
            rows.at[0].at[pl.ds(0, wb)], agg.at[pl.ds(rbase + off, wb)]
        )
        off += wb
    plsc.subcore_barrier()

    def _unpack(j, pb):
        row = lax.shift_right_logical(j, 1)
        o = (j & 1) * CHUNK
        for k in range(CHUNK // 16):
            v = idx_p[row, pl.ds(o + k * 16, 16)]
            srcb[pb, pl.ds(k * 16, 16)] = v & ((1 << _DST_SHIFT) - 1)
            dstb[pb, pl.ds(k * 16, 16)] = lax.shift_right_logical(v, _DST_SHIFT)

    for b in range(NB - 1):
        _unpack(jnp.int32(b), b)
        pltpu.async_copy(x_hbm.at[srcb.at[b]], rows.at[b], gsems[b])

    def _outer(t, _):
        for b in range(NB):
            j = t * NB + b
            pltpu.make_async_copy(x_hbm.at[srcb.at[b]], rows.at[b], gsems[b]).wait()
            nxt = (b + NB - 1) % NB

            @pl.when(j + NB - 1 < steps)
            def _():
                _unpack(j + NB - 1, nxt)
                pltpu.async_copy(x_hbm.at[srcb.at[nxt]], rows.at[nxt], gsems[nxt])

            pltpu.sync_copy(rows.at[b], agg.at[dstb.at[b]], add=True)
        return 0

    lax.fori_loop(0, steps // NB, _outer, 0)
    plsc.subcore_barrier()

    off = 0
    for k, wb in enumerate(_WB):
        r0 = rbase + off
        b = k % NB
        pltpu.sync_copy(agg.at[pl.ds(r0, wb)], rows.at[b].at[pl.ds(0, wb)])
        pltpu.sync_copy(rows.at[b].at[pl.ds(0, wb)], out_hbm.at[c, pl.ds(r0, wb)])
        off += wb


_sc_agg = functools.partial(
    pl.kernel,
    out_type=jax.ShapeDtypeStruct((NC, N_PAD, D), jnp.float32),
    mesh=plsc.VectorSubcoreMesh(core_axis_name="c", subcore_axis_name="s"),
    scratch_types=[
        pltpu.VMEM((IDX_MAX, LANES), jnp.int32),
        pltpu.VMEM((NB, CHUNK), jnp.int32),
        pltpu.VMEM((NB, CHUNK), jnp.int32),
        pltpu.VMEM((NB, CHUNK, D), jnp.float32),
        pltpu.VMEM_SHARED((N_PAD, D), jnp.float32),
        pltpu.SemaphoreType.DMA,
        pltpu.SemaphoreType.DMA,
        pltpu.SemaphoreType.DMA,
        pltpu.SemaphoreType.DMA,
    ],
)(_sc_body)


def _tc_body(x_ref, p_ref, w_ref, m_ref, o_ref):
    self_term = jnp.dot(x_ref[...], w_ref[...], preferred_element_type=jnp.float32)
    agg = p_ref[0] + p_ref[1]
    neigh = jnp.dot(agg, m_ref[...], preferred_element_type=jnp.float32)
    o_ref[...] = jnp.maximum(self_term + neigh, 0.0)


_TC_BLK = 1000


def _tc_combine(x, partials, W, M):
    return pl.pallas_call(
        _tc_body,
        grid=(N_NODES // _TC_BLK,),
        in_specs=[
            pl.BlockSpec((_TC_BLK, D), lambda i: (i, 0)),
            pl.BlockSpec((NC, _TC_BLK, D), lambda i: (0, i, 0)),
            pl.BlockSpec((D, D), lambda i: (0, 0)),
            pl.BlockSpec((D, D), lambda i: (0, 0)),
        ],
        out_specs=pl.BlockSpec((_TC_BLK, D), lambda i: (i, 0)),
        out_shape=jax.ShapeDtypeStruct((N_NODES, D), jnp.float32),
    )(x, partials, W, M)


@jax.jit
def kernel(x, edge_index, W, M):
    src = edge_index[0].astype(jnp.int32)
    dst = edge_index[1].astype(jnp.int32)
    pad = E_PAD - N_EDGES
    packed = src | (dst << _DST_SHIFT)
    pad_val = jnp.int32((N_PAD - 1) << _DST_SHIFT)
    packed = jnp.concatenate([packed, jnp.full((pad,), pad_val, jnp.int32)])
    partials = _sc_agg(x, packed.reshape(-1, LANES))
    return _tc_combine(x, partials, W, M)


# speedup vs baseline: 1.1252x; 1.1252x over previous
import functools

import jax
import jax.numpy as jnp
from jax import lax
from jax.experimental import pallas as pl
from jax.experimental.pallas import tpu as pltpu
from jax.experimental.pallas import tpu_sc as plsc

N_NODES = 10000
N_EDGES = 320000
D = 128

NC = 2
NS = 16
NW = NC * NS

N_PAD = 10112
ROWS_PER_TILE = N_PAD // NS
LANES = 128
CHUNK = 64
NB = 4
S0 = 256
S1 = 64
E_PAD = (S0 + S1) * CHUNK * NS
IDX_MAX = max(S0, S1) * CHUNK // LANES
_WB = [CHUNK] * 9 + [ROWS_PER_TILE - 9 * CHUNK]

_DST_SHIFT = 14


def _sc_body(x_hbm, packed_hbm, out_hbm, idx_p, srcb, dstb, rows, agg, *gsems):
    c = lax.axis_index("c")
    s = lax.axis_index("s")

    steps = jnp.where(c == 0, S0, S1)
    idx_rows = lax.shift_right_logical(steps, 1)
    base = jnp.where(c == 0, s * (S0 // 2), NS * (S0 // 2) + s * (S1 // 2))

    for q in range(IDX_MAX // 32):

        @pl.when(q * 32 < idx_rows)
        def _():
            pltpu.sync_copy(
                packed_hbm.at[pl.ds(base + q * 32, 32)],
                idx_p.at[pl.ds(q * 32, 32)],
            )

    z = jnp.zeros((16,), jnp.float32)

    def _zero_row(i, _):
        for k in range(8):
            rows[0, i, pl.ds(k * 16, 16)] = z
        return 0

    lax.fori_loop(0, CHUNK, _zero_row, 0)
    rbase = s * ROWS_PER_TILE
    off = 0
    for wb in _WB:
        pltpu.sync_copy(
            rows.at[0].at[pl.ds(0, wb)], agg.at[pl.ds(rbase + off, wb)]
        )
        off += wb
    plsc.subcore_barrier()

    def _unpack(j, pb):
        row = lax.shift_right_logical(j, 1)
        o = (j & 1) * CHUNK
        for k in range(CHUNK // 16):
            v = idx_p[row, pl.ds(o + k * 16, 16)]
            srcb[pb, pl.ds(k * 16, 16)] = v & ((1 << _DST_SHIFT) - 1)
            dstb[pb, pl.ds(k * 16, 16)] = lax.shift_right_logical(v, _DST_SHIFT)

    for b in range(NB - 1):
        _unpack(jnp.int32(b), b)
        pltpu.async_copy(x_hbm.at[srcb.at[b]], rows.at[b], gsems[b])

    def _outer(t, _):
        for b in range(NB):
            j = t * NB + b
            pltpu.make_async_copy(x_hbm.at[srcb.at[b]], rows.at[b], gsems[b]).wait()
            nxt = (b + NB - 1) % NB

            @pl.when(j + NB - 1 < steps)
            def _():
                _unpack(j + NB - 1, nxt)
                pltpu.async_copy(x_hbm.at[srcb.at[nxt]], rows.at[nxt], gsems[nxt])

            pltpu.sync_copy(rows.at[b], agg.at[dstb.at[b]], add=True)
        return 0

    lax.fori_loop(0, steps // NB, _outer, 0)
    plsc.subcore_barrier()

    off = 0
    for k, wb in enumerate(_WB):
        r0 = rbase + off
        b = k % NB
        pltpu.sync_copy(agg.at[pl.ds(r0, wb)], rows.at[b].at[pl.ds(0, wb)])
        pltpu.sync_copy(rows.at[b].at[pl.ds(0, wb)], out_hbm.at[c, pl.ds(r0, wb)])
        off += wb


_sc_agg = functools.partial(
    pl.kernel,
    out_type=jax.ShapeDtypeStruct((NC, N_PAD, D), jnp.float32),
    mesh=plsc.VectorSubcoreMesh(core_axis_name="c", subcore_axis_name="s"),
    scratch_types=[
        pltpu.VMEM((IDX_MAX, LANES), jnp.int32),
        pltpu.VMEM((NB, CHUNK), jnp.int32),
        pltpu.VMEM((NB, CHUNK), jnp.int32),
        pltpu.VMEM((NB, CHUNK, D), jnp.float32),
        pltpu.VMEM_SHARED((N_PAD, D), jnp.float32),
        pltpu.SemaphoreType.DMA,
        pltpu.SemaphoreType.DMA,
        pltpu.SemaphoreType.DMA,
        pltpu.SemaphoreType.DMA,
    ],
)(_sc_body)


def _tc_body(x_ref, p_ref, w_ref, m_ref, o_ref):
    self_term = jnp.dot(x_ref[...], w_ref[...], preferred_element_type=jnp.float32)
    agg = p_ref[0] + p_ref[1]
    neigh = jnp.dot(agg, m_ref[...], preferred_element_type=jnp.float32)
    o_ref[...] = jnp.maximum(self_term + neigh, 0.0)


_TC_BLK = 1000


def _tc_combine(x, partials, W, M):
    return pl.pallas_call(
        _tc_body,
        grid=(N_NODES // _TC_BLK,),
        in_specs=[
            pl.BlockSpec((_TC_BLK, D), lambda i: (i, 0)),
            pl.BlockSpec((NC, _TC_BLK, D), lambda i: (0, i, 0)),
            pl.BlockSpec((D, D), lambda i: (0, 0)),
            pl.BlockSpec((D, D), lambda i: (0, 0)),
        ],
        out_specs=pl.BlockSpec((_TC_BLK, D), lambda i: (i, 0)),
        out_shape=jax.ShapeDtypeStruct((N_NODES, D), jnp.float32),
    )(x, partials, W, M)


@jax.jit
def kernel(x, edge_index, W, M):
    src = edge_index[0].astype(jnp.int32)
    dst = edge_index[1].astype(jnp.int32)
    pad = E_PAD - N_EDGES
    packed = src | (dst << _DST_SHIFT)
    pad_val = jnp.int32((N_PAD - 1) << _DST_SHIFT)
    packed = jnp.concatenate([packed, jnp.full((pad,), pad_val, jnp.int32)])
    partials = _sc_agg(x, packed.reshape(-1, LANES))
    return _tc_combine(x, partials, W, M)
